# baseline (speedup 1.0000x reference)
import functools

import jax
import jax.numpy as jnp
from jax import lax
from jax.experimental import pallas as pl
from jax.experimental.pallas import tpu as pltpu
from jax.experimental.pallas import tpu_sc as plsc

BIN = 0.5
MINR = 0.5
NBINS = 10
NT = 9
D = 32
L = 16
IDX_CHUNK = 128
NW = 32


def _ordrec_body(b_per_w, n_chunks,
                 uid_hbm, iid_hbm, uf_hbm, vf_hbm, ub_hbm, vb_hbm,
                 ut1_hbm, ubeta_hbm,
                 mass_hbm, mean_hbm, mode_hbm, edges_hbm,
                 uidx_v, iidx_v, uf_v, vf_v, ub_v, vb_v, ut1_v, ubeta_v,
                 mass_v, mean_v, mode_v, edges_v, sem):
    cid = lax.axis_index("c")
    sid = lax.axis_index("s")
    wid = sid * 2 + cid
    base = pl.multiple_of(wid * b_per_w, b_per_w)

    for j in range(n_chunks):
        pltpu.sync_copy(uid_hbm.at[pl.ds(base + j * IDX_CHUNK, IDX_CHUNK)],
                        uidx_v.at[j])
        pltpu.sync_copy(iid_hbm.at[pl.ds(base + j * IDX_CHUNK, IDX_CHUNK)],
                        iidx_v.at[j])

    copies = [
        pltpu.async_copy(ub_hbm.at[pl.ds(base, b_per_w)], ub_v, sem),
        pltpu.async_copy(vb_hbm.at[pl.ds(base, b_per_w)], vb_v, sem),
        pltpu.async_copy(ut1_hbm.at[pl.ds(base, b_per_w)], ut1_v, sem),
    ]
    for j in range(n_chunks):
        sl = pl.ds(j * IDX_CHUNK, IDX_CHUNK)
        uj = uidx_v.at[j]
        ij = iidx_v.at[j]
        copies.append(pltpu.async_copy(uf_hbm.at[uj], uf_v.at[sl], sem))
        copies.append(pltpu.async_copy(vf_hbm.at[ij], vf_v.at[sl], sem))
        copies.append(pltpu.async_copy(ubeta_hbm.at[uj], ubeta_v.at[sl], sem))
    for c in copies:
        c.wait()

    def group(g, _):
        off = pl.multiple_of(g * L, L)
        rows = off + lax.iota(jnp.int32, 16)

        acc = jnp.zeros((L,), jnp.float32)
        for d in range(D):
            di = jnp.full((L,), d, jnp.int32)
            a = plsc.load_gather(uf_v, [rows, di])
            b = plsc.load_gather(vf_v, [rows, di])
            acc = acc + a * b

        ub = ub_v[pl.ds(off, L)]
        vb = vb_v[pl.ds(off, L)]
        ut1 = ut1_v[pl.ds(off, L)]

        y = acc + vb + ub

        T = [ut1]
        for k in range(NT - 1):
            bk = plsc.load_gather(ubeta_v, [rows, jnp.full((L,), k, jnp.int32)])
            T.append(T[-1] + jnp.exp(bk))

        one = jnp.ones((L,), jnp.float32)
        sig = [one / (one + jnp.exp(y - t)) for t in T]

        mass = [sig[0]]
        for k in range(1, NT):
            mass.append(sig[k] - sig[k - 1])
        mass.append(one - sig[NT - 1])

        mean = jnp.zeros((L,), jnp.float32)
        best = mass[0]
        bestk = jnp.zeros((L,), jnp.float32)
        for k in range(NBINS):
            mean = mean + mass[k] * (MINR + k * BIN)
            if k > 0:
                gt = mass[k] > best
                best = jnp.where(gt, mass[k], best)
                bestk = jnp.where(gt, jnp.full((L,), float(k), jnp.float32),
                                  bestk)
        mode = MINR + bestk * BIN

        for k in range(NT):
            plsc.store_scatter(edges_v, [rows, jnp.full((L,), k, jnp.int32)],
                               T[k])
        plsc.store_scatter(edges_v, [rows, jnp.full((L,), NT, jnp.int32)],
                           jnp.full((L,), jnp.inf, jnp.float32))
        for k in range(NBINS):
            plsc.store_scatter(mass_v, [rows, jnp.full((L,), k, jnp.int32)],
                               mass[k])
        mean_v[pl.ds(off, L)] = mean
        mode_v[pl.ds(off, L)] = mode
        return 0

    lax.fori_loop(0, b_per_w // L, group, 0)

    pltpu.sync_copy(mass_v, mass_hbm.at[pl.ds(base, b_per_w)])
    pltpu.sync_copy(mean_v, mean_hbm.at[pl.ds(base, b_per_w)])
    pltpu.sync_copy(mode_v, mode_hbm.at[pl.ds(base, b_per_w)])
    pltpu.sync_copy(edges_v, edges_hbm.at[pl.ds(base, b_per_w)])


def kernel(uid_input, iid_input, uid_features, iid_features, uid_bias,
           iid_bias, uid_t1, iid_t1, uid_beta, iid_beta):
    del iid_t1, iid_beta
    B = uid_input.shape[0]
    b_per_w = B // NW
    n_chunks = b_per_w // IDX_CHUNK
    f32 = jnp.float32
    mesh = plsc.VectorSubcoreMesh(core_axis_name="c", subcore_axis_name="s")

    run = pl.kernel(
        functools.partial(_ordrec_body, b_per_w, n_chunks),
        mesh=mesh,
        out_type=[
            jax.ShapeDtypeStruct((B, NBINS), f32),
            jax.ShapeDtypeStruct((B,), f32),
            jax.ShapeDtypeStruct((B,), f32),
            jax.ShapeDtypeStruct((B, NBINS), f32),
        ],
        scratch_types=[
            pltpu.VMEM((n_chunks, IDX_CHUNK), jnp.int32),
            pltpu.VMEM((n_chunks, IDX_CHUNK), jnp.int32),
            pltpu.VMEM((b_per_w, D), f32),
            pltpu.VMEM((b_per_w, D), f32),
            pltpu.VMEM((b_per_w,), f32),
            pltpu.VMEM((b_per_w,), f32),
            pltpu.VMEM((b_per_w,), f32),
            pltpu.VMEM((b_per_w, NT - 1), f32),
            pltpu.VMEM((b_per_w, NBINS), f32),
            pltpu.VMEM((b_per_w,), f32),
            pltpu.VMEM((b_per_w,), f32),
            pltpu.VMEM((b_per_w, NBINS), f32),
            pltpu.SemaphoreType.DMA,
        ],
        compiler_params=pltpu.CompilerParams(
            use_tc_tiling_on_sc=False, needs_layout_passes=False),
    )
    ub_g = jnp.take(uid_bias, uid_input, axis=0).reshape(-1)
    vb_g = jnp.take(iid_bias, iid_input, axis=0).reshape(-1)
    ut1_g = jnp.take(uid_t1, uid_input, axis=0).reshape(-1)
    bins_mass, bins_mean, bins_mode, edges = run(
        uid_input, iid_input, uid_features, iid_features,
        ub_g, vb_g, ut1_g, uid_beta)
    return bins_mass, bins_mean, bins_mode, edges


# speedup vs baseline: 4.5032x; 4.5032x over previous
import functools

import jax
import jax.numpy as jnp
from jax import lax
from jax.experimental import pallas as pl
from jax.experimental.pallas import tpu as pltpu
from jax.experimental.pallas import tpu_sc as plsc

BIN = 0.5
MINR = 0.5
NBINS = 10
NT = 9
D = 32
L = 16
NW = 32


def _ordrec_body(b_per_w,
                 uf_hbm, vf_hbm, ub_hbm, vb_hbm, ut1_hbm, ubeta_hbm,
                 mass_hbm, mean_hbm, mode_hbm, edges_hbm,
                 uf_v, vf_v, ub_v, vb_v, ut1_v, ubeta_v,
                 mass_v, mean_v, mode_v, edges_v, sem):
    cid = lax.axis_index("c")
    sid = lax.axis_index("s")
    wid = sid * 2 + cid
    base = pl.multiple_of(wid * b_per_w, b_per_w)
    bsl = pl.ds(base, b_per_w)

    copies = [
        pltpu.async_copy(uf_hbm.at[bsl], uf_v, sem),
        pltpu.async_copy(vf_hbm.at[bsl], vf_v, sem),
        pltpu.async_copy(ub_hbm.at[bsl], ub_v, sem),
        pltpu.async_copy(vb_hbm.at[bsl], vb_v, sem),
        pltpu.async_copy(ut1_hbm.at[bsl], ut1_v, sem),
        pltpu.async_copy(ubeta_hbm.at[bsl], ubeta_v, sem),
    ]
    for c in copies:
        c.wait()

    def group(g, _):
        off = pl.multiple_of(g * L, L)
        rows = off + lax.iota(jnp.int32, 16)

        acc = jnp.zeros((L,), jnp.float32)
        for d in range(D):
            di = jnp.full((L,), d, jnp.int32)
            a = plsc.load_gather(uf_v, [rows, di])
            b = plsc.load_gather(vf_v, [rows, di])
            acc = acc + a * b

        ub = ub_v[pl.ds(off, L)]
        vb = vb_v[pl.ds(off, L)]
        ut1 = ut1_v[pl.ds(off, L)]

        y = acc + vb + ub

        T = [ut1]
        for k in range(NT - 1):
            bk = plsc.load_gather(ubeta_v, [rows, jnp.full((L,), k, jnp.int32)])
            T.append(T[-1] + jnp.exp(bk))

        one = jnp.ones((L,), jnp.float32)
        sig = [one / (one + jnp.exp(y - t)) for t in T]

        mass = [sig[0]]
        for k in range(1, NT):
            mass.append(sig[k] - sig[k - 1])
        mass.append(one - sig[NT - 1])

        mean = jnp.zeros((L,), jnp.float32)
        best = mass[0]
        bestk = jnp.zeros((L,), jnp.float32)
        for k in range(NBINS):
            mean = mean + mass[k] * (MINR + k * BIN)
            if k > 0:
                gt = mass[k] > best
                best = jnp.where(gt, mass[k], best)
                bestk = jnp.where(gt, jnp.full((L,), float(k), jnp.float32),
                                  bestk)
        mode = MINR + bestk * BIN

        for k in range(NT):
            plsc.store_scatter(edges_v, [rows, jnp.full((L,), k, jnp.int32)],
                               T[k])
        plsc.store_scatter(edges_v, [rows, jnp.full((L,), NT, jnp.int32)],
                           jnp.full((L,), jnp.inf, jnp.float32))
        for k in range(NBINS):
            plsc.store_scatter(mass_v, [rows, jnp.full((L,), k, jnp.int32)],
                               mass[k])
        mean_v[pl.ds(off, L)] = mean
        mode_v[pl.ds(off, L)] = mode
        return 0

    lax.fori_loop(0, b_per_w // L, group, 0)

    pltpu.sync_copy(mass_v, mass_hbm.at[bsl])
    pltpu.sync_copy(mean_v, mean_hbm.at[bsl])
    pltpu.sync_copy(mode_v, mode_hbm.at[bsl])
    pltpu.sync_copy(edges_v, edges_hbm.at[bsl])


def kernel(uid_input, iid_input, uid_features, iid_features, uid_bias,
           iid_bias, uid_t1, iid_t1, uid_beta, iid_beta):
    del iid_t1, iid_beta
    B = uid_input.shape[0]
    b_per_w = B // NW
    f32 = jnp.float32
    mesh = plsc.VectorSubcoreMesh(core_axis_name="c", subcore_axis_name="s")

    uf_g = jnp.take(uid_features, uid_input, axis=0)
    vf_g = jnp.take(iid_features, iid_input, axis=0)
    ub_g = jnp.take(uid_bias, uid_input, axis=0).reshape(-1)
    vb_g = jnp.take(iid_bias, iid_input, axis=0).reshape(-1)
    ut1_g = jnp.take(uid_t1, uid_input, axis=0).reshape(-1)
    ubeta_g = jnp.take(uid_beta, uid_input, axis=0)

    run = pl.kernel(
        functools.partial(_ordrec_body, b_per_w),
        mesh=mesh,
        out_type=[
            jax.ShapeDtypeStruct((B, NBINS), f32),
            jax.ShapeDtypeStruct((B,), f32),
            jax.ShapeDtypeStruct((B,), f32),
            jax.ShapeDtypeStruct((B, NBINS), f32),
        ],
        scratch_types=[
            pltpu.VMEM((b_per_w, D), f32),
            pltpu.VMEM((b_per_w, D), f32),
            pltpu.VMEM((b_per_w,), f32),
            pltpu.VMEM((b_per_w,), f32),
            pltpu.VMEM((b_per_w,), f32),
            pltpu.VMEM((b_per_w, NT - 1), f32),
            pltpu.VMEM((b_per_w, NBINS), f32),
            pltpu.VMEM((b_per_w,), f32),
            pltpu.VMEM((b_per_w,), f32),
            pltpu.VMEM((b_per_w, NBINS), f32),
            pltpu.SemaphoreType.DMA,
        ],
        compiler_params=pltpu.CompilerParams(
            use_tc_tiling_on_sc=False, needs_layout_passes=False),
    )
    bins_mass, bins_mean, bins_mode, edges = run(
        uf_g, vf_g, ub_g, vb_g, ut1_g, ubeta_g)
    return bins_mass, bins_mean, bins_mode, edges


# speedup vs baseline: 4.6931x; 1.0422x over previous
import functools

import jax
import jax.numpy as jnp
from jax import lax
from jax.experimental import pallas as pl
from jax.experimental.pallas import tpu as pltpu
from jax.experimental.pallas import tpu_sc as plsc

BIN = 0.5
MINR = 0.5
NBINS = 10
NT = 9
D = 32
L = 16
IDX_CHUNK = 128
NW = 32


def _ordrec_body(b_per_w, n_chunks,
                 uid_hbm, iid_hbm, uf_hbm, vf_hbm, ub_hbm, vb_hbm,
                 ut1_hbm, ubeta_hbm,
                 mass_hbm, mean_hbm, mode_hbm, edges_hbm,
                 uidx_v, iidx_v, uf_v, vf_v, ub_v, vb_v, ut1_v, ubeta_v,
                 mass_v, mean_v, mode_v, edges_v, sem):
    cid = lax.axis_index("c")
    sid = lax.axis_index("s")
    wid = sid * 2 + cid
    base = pl.multiple_of(wid * b_per_w, b_per_w)
    bsl = pl.ds(base, b_per_w)

    for j in range(n_chunks):
        pltpu.sync_copy(uid_hbm.at[pl.ds(base + j * IDX_CHUNK, IDX_CHUNK)],
                        uidx_v.at[j])
        pltpu.sync_copy(iid_hbm.at[pl.ds(base + j * IDX_CHUNK, IDX_CHUNK)],
                        iidx_v.at[j])

    copies = [
        pltpu.async_copy(uf_hbm.at[bsl], uf_v, sem),
        pltpu.async_copy(vf_hbm.at[bsl], vf_v, sem),
        pltpu.async_copy(ubeta_hbm.at[bsl], ubeta_v, sem),
    ]
    for j in range(n_chunks):
        sl = pl.ds(j * IDX_CHUNK, IDX_CHUNK)
        uj = uidx_v.at[j]
        ij = iidx_v.at[j]
        copies.append(pltpu.async_copy(ub_hbm.at[uj], ub_v.at[sl], sem))
        copies.append(pltpu.async_copy(vb_hbm.at[ij], vb_v.at[sl], sem))
        copies.append(pltpu.async_copy(ut1_hbm.at[uj], ut1_v.at[sl], sem))
    for c in copies:
        c.wait()

    def group(g, _):
        off = pl.multiple_of(g * L, L)
        rows = off + lax.iota(jnp.int32, 16)

        acc = jnp.zeros((L,), jnp.float32)
        for d in range(D):
            di = jnp.full((L,), d, jnp.int32)
            a = plsc.load_gather(uf_v, [rows, di])
            b = plsc.load_gather(vf_v, [rows, di])
            acc = acc + a * b

        ub = ub_v[pl.ds(off, L)]
        vb = vb_v[pl.ds(off, L)]
        ut1 = ut1_v[pl.ds(off, L)]

        y = acc + vb + ub

        T = [ut1]
        for k in range(NT - 1):
            bk = plsc.load_gather(ubeta_v, [rows, jnp.full((L,), k, jnp.int32)])
            T.append(T[-1] + jnp.exp(bk))

        one = jnp.ones((L,), jnp.float32)
        sig = [one / (one + jnp.exp(y - t)) for t in T]

        mass = [sig[0]]
        for k in range(1, NT):
            mass.append(sig[k] - sig[k - 1])
        mass.append(one - sig[NT - 1])

        mean = jnp.zeros((L,), jnp.float32)
        best = mass[0]
        bestk = jnp.zeros((L,), jnp.float32)
        for k in range(NBINS):
            mean = mean + mass[k] * (MINR + k * BIN)
            if k > 0:
                gt = mass[k] > best
                best = jnp.where(gt, mass[k], best)
                bestk = jnp.where(gt, jnp.full((L,), float(k), jnp.float32),
                                  bestk)
        mode = MINR + bestk * BIN

        for k in range(NT):
            plsc.store_scatter(edges_v, [rows, jnp.full((L,), k, jnp.int32)],
                               T[k])
        plsc.store_scatter(edges_v, [rows, jnp.full((L,), NT, jnp.int32)],
                           jnp.full((L,), jnp.inf, jnp.float32))
        for k in range(NBINS):
            plsc.store_scatter(mass_v, [rows, jnp.full((L,), k, jnp.int32)],
                               mass[k])
        mean_v[pl.ds(off, L)] = mean
        mode_v[pl.ds(off, L)] = mode
        return 0

    lax.fori_loop(0, b_per_w // L, group, 0)

    pltpu.sync_copy(mass_v, mass_hbm.at[bsl])
    pltpu.sync_copy(mean_v, mean_hbm.at[bsl])
    pltpu.sync_copy(mode_v, mode_hbm.at[bsl])
    pltpu.sync_copy(edges_v, edges_hbm.at[bsl])


def kernel(uid_input, iid_input, uid_features, iid_features, uid_bias,
           iid_bias, uid_t1, iid_t1, uid_beta, iid_beta):
    del iid_t1, iid_beta
    B = uid_input.shape[0]
    b_per_w = B // NW
    n_chunks = b_per_w // IDX_CHUNK
    f32 = jnp.float32
    mesh = plsc.VectorSubcoreMesh(core_axis_name="c", subcore_axis_name="s")

    uf_g = jnp.take(uid_features, uid_input, axis=0)
    vf_g = jnp.take(iid_features, iid_input, axis=0)
    ubeta_g = jnp.take(uid_beta, uid_input, axis=0)

    run = pl.kernel(
        functools.partial(_ordrec_body, b_per_w, n_chunks),
        mesh=mesh,
        out_type=[
            jax.ShapeDtypeStruct((B, NBINS), f32),
            jax.ShapeDtypeStruct((B,), f32),
            jax.ShapeDtypeStruct((B,), f32),
            jax.ShapeDtypeStruct((B, NBINS), f32),
        ],
        scratch_types=[
            pltpu.VMEM((n_chunks, IDX_CHUNK), jnp.int32),
            pltpu.VMEM((n_chunks, IDX_CHUNK), jnp.int32),
            pltpu.VMEM((b_per_w, D), f32),
            pltpu.VMEM((b_per_w, D), f32),
            pltpu.VMEM((b_per_w,), f32),
            pltpu.VMEM((b_per_w,), f32),
            pltpu.VMEM((b_per_w,), f32),
            pltpu.VMEM((b_per_w, NT - 1), f32),
            pltpu.VMEM((b_per_w, NBINS), f32),
            pltpu.VMEM((b_per_w,), f32),
            pltpu.VMEM((b_per_w,), f32),
            pltpu.VMEM((b_per_w, NBINS), f32),
            pltpu.SemaphoreType.DMA,
        ],
        compiler_params=pltpu.CompilerParams(
            use_tc_tiling_on_sc=False, needs_layout_passes=False),
    )
    bins_mass, bins_mean, bins_mode, edges = run(
        uid_input, iid_input, uf_g, vf_g,
        uid_bias.reshape(-1), iid_bias.reshape(-1), uid_t1.reshape(-1),
        ubeta_g)
    return bins_mass, bins_mean, bins_mode, edges


# speedup vs baseline: 5.7984x; 1.2355x over previous
import functools

import jax
import jax.numpy as jnp
from jax import lax
from jax.experimental import pallas as pl
from jax.experimental.pallas import tpu as pltpu
from jax.experimental.pallas import tpu_sc as plsc

BIN = 0.5
MINR = 0.5
NBINS = 10
NT = 9
D = 32
L = 16
IDX_CHUNK = 128
NW = 32


def _ordrec_body(b_per_w, n_chunks,
                 uid_hbm, iid_hbm, uf_hbm, vf_hbm, ub_hbm, vb_hbm,
                 ut1_hbm, ubeta_hbm,
                 mass_hbm, mean_hbm, mode_hbm, edges_hbm,
                 uidx_v, iidx_v, uf_v, vf_v, ub_v, vb_v, ut1_v, ubeta_v,
                 mass_v, mean_v, mode_v, edges_v, sem):
    cid = lax.axis_index("c")
    sid = lax.axis_index("s")
    wid = sid * 2 + cid
    base = pl.multiple_of(wid * b_per_w, b_per_w)
    bsl = pl.ds(base, b_per_w)

    for j in range(n_chunks):
        pltpu.sync_copy(uid_hbm.at[pl.ds(base + j * IDX_CHUNK, IDX_CHUNK)],
                        uidx_v.at[j])
        pltpu.sync_copy(iid_hbm.at[pl.ds(base + j * IDX_CHUNK, IDX_CHUNK)],
                        iidx_v.at[j])

    copies = [
        pltpu.async_copy(uf_hbm.at[bsl], uf_v, sem),
        pltpu.async_copy(vf_hbm.at[bsl], vf_v, sem),
        pltpu.async_copy(ubeta_hbm.at[bsl], ubeta_v, sem),
    ]
    for j in range(n_chunks):
        sl = pl.ds(j * IDX_CHUNK, IDX_CHUNK)
        uj = uidx_v.at[j]
        ij = iidx_v.at[j]
        copies.append(pltpu.async_copy(ub_hbm.at[uj], ub_v.at[sl], sem))
        copies.append(pltpu.async_copy(vb_hbm.at[ij], vb_v.at[sl], sem))
        copies.append(pltpu.async_copy(ut1_hbm.at[uj], ut1_v.at[sl], sem))
    for c in copies:
        c.wait()

    def group(g, _):
        off = pl.multiple_of(g * L, L)
        rows = off + lax.iota(jnp.int32, 16)

        acc = jnp.zeros((L,), jnp.float32)
        for d in range(D):
            di = jnp.full((L,), d, jnp.int32)
            a = plsc.load_gather(uf_v, [rows, di])
            b = plsc.load_gather(vf_v, [rows, di])
            acc = acc + a * b

        ub = ub_v[pl.ds(off, L)]
        vb = vb_v[pl.ds(off, L)]
        ut1 = ut1_v[pl.ds(off, L)]

        y = acc + vb + ub

        T = [ut1]
        for k in range(NT - 1):
            bk = plsc.load_gather(ubeta_v, [rows, jnp.full((L,), k, jnp.int32)])
            T.append(T[-1] + jnp.exp(bk))

        one = jnp.ones((L,), jnp.float32)
        sig = [one / (one + jnp.exp(y - t)) for t in T]

        mass = [sig[0]]
        for k in range(1, NT):
            mass.append(sig[k] - sig[k - 1])
        mass.append(one - sig[NT - 1])

        mean = jnp.zeros((L,), jnp.float32)
        best = mass[0]
        bestk = jnp.zeros((L,), jnp.float32)
        for k in range(NBINS):
            mean = mean + mass[k] * (MINR + k * BIN)
            if k > 0:
                gt = mass[k] > best
                best = jnp.where(gt, mass[k], best)
                bestk = jnp.where(gt, jnp.full((L,), float(k), jnp.float32),
                                  bestk)
        mode = MINR + bestk * BIN

        for k in range(NT):
            plsc.store_scatter(edges_v, [rows, jnp.full((L,), k, jnp.int32)],
                               T[k])
        plsc.store_scatter(edges_v, [rows, jnp.full((L,), NT, jnp.int32)],
                           jnp.full((L,), jnp.inf, jnp.float32))
        for k in range(NBINS):
            plsc.store_scatter(mass_v, [rows, jnp.full((L,), k, jnp.int32)],
                               mass[k])
        mean_v[pl.ds(off, L)] = mean
        mode_v[pl.ds(off, L)] = mode
        return 0

    lax.fori_loop(0, b_per_w // L, group, 0)

    pltpu.sync_copy(mass_v, mass_hbm.at[bsl])
    pltpu.sync_copy(mean_v, mean_hbm.at[bsl])
    pltpu.sync_copy(mode_v, mode_hbm.at[bsl])
    pltpu.sync_copy(edges_v, edges_hbm.at[bsl])


def kernel(uid_input, iid_input, uid_features, iid_features, uid_bias,
           iid_bias, uid_t1, iid_t1, uid_beta, iid_beta):
    del iid_t1, iid_beta
    B = uid_input.shape[0]
    b_per_w = B // NW
    n_chunks = b_per_w // IDX_CHUNK
    f32 = jnp.float32
    mesh = plsc.VectorSubcoreMesh(core_axis_name="c", subcore_axis_name="s")

    uf_g = uid_features.at[uid_input].get(mode="promise_in_bounds")
    vf_g = iid_features.at[iid_input].get(mode="promise_in_bounds")
    ubeta_g = uid_beta.at[uid_input].get(mode="promise_in_bounds")

    run = pl.kernel(
        functools.partial(_ordrec_body, b_per_w, n_chunks),
        mesh=mesh,
        out_type=[
            jax.ShapeDtypeStruct((B, NBINS), f32),
            jax.ShapeDtypeStruct((B,), f32),
            jax.ShapeDtypeStruct((B,), f32),
            jax.ShapeDtypeStruct((B, NBINS), f32),
        ],
        scratch_types=[
            pltpu.VMEM((n_chunks, IDX_CHUNK), jnp.int32),
            pltpu.VMEM((n_chunks, IDX_CHUNK), jnp.int32),
            pltpu.VMEM((b_per_w, D), f32),
            pltpu.VMEM((b_per_w, D), f32),
            pltpu.VMEM((b_per_w,), f32),
            pltpu.VMEM((b_per_w,), f32),
            pltpu.VMEM((b_per_w,), f32),
            pltpu.VMEM((b_per_w, NT - 1), f32),
            pltpu.VMEM((b_per_w, NBINS), f32),
            pltpu.VMEM((b_per_w,), f32),
            pltpu.VMEM((b_per_w,), f32),
            pltpu.VMEM((b_per_w, NBINS), f32),
            pltpu.SemaphoreType.DMA,
        ],
        compiler_params=pltpu.CompilerParams(
            use_tc_tiling_on_sc=False, needs_layout_passes=False),
    )
    bins_mass, bins_mean, bins_mode, edges = run(
        uid_input, iid_input, uf_g, vf_g,
        uid_bias.reshape(-1), iid_bias.reshape(-1), uid_t1.reshape(-1),
        ubeta_g)
    return bins_mass, bins_mean, bins_mode, edges


# speedup vs baseline: 5.8888x; 1.0156x over previous
import functools

import jax
import jax.numpy as jnp
from jax import lax
from jax.experimental import pallas as pl
from jax.experimental.pallas import tpu as pltpu
from jax.experimental.pallas import tpu_sc as plsc

BIN = 0.5
MINR = 0.5
NBINS = 10
NT = 9
D = 32
L = 16
IDX_CHUNK = 128
NW = 32


def _ordrec_body(b_per_w, n_chunks,
                 uid_hbm, iid_hbm, uf_hbm, vf_hbm, ub_hbm, vb_hbm,
                 ut1_hbm, ubeta_hbm,
                 mass_hbm, mean_hbm, mode_hbm, edges_hbm,
                 uidx_v, iidx_v, uf_v, vf_v, ub_v, vb_v, ut1_v, ubeta_v,
                 mass_v, mean_v, mode_v, edges_v, sem, sem2):
    cid = lax.axis_index("c")
    sid = lax.axis_index("s")
    wid = sid * 2 + cid
    base = pl.multiple_of(wid * b_per_w, b_per_w)
    bsl = pl.ds(base, b_per_w)

    idx_copies = []
    for j in range(n_chunks):
        idx_copies.append(pltpu.async_copy(
            uid_hbm.at[pl.ds(base + j * IDX_CHUNK, IDX_CHUNK)],
            uidx_v.at[j], sem2))
        idx_copies.append(pltpu.async_copy(
            iid_hbm.at[pl.ds(base + j * IDX_CHUNK, IDX_CHUNK)],
            iidx_v.at[j], sem2))

    copies = [
        pltpu.async_copy(uf_hbm.at[bsl], uf_v, sem),
        pltpu.async_copy(vf_hbm.at[bsl], vf_v, sem),
        pltpu.async_copy(ubeta_hbm.at[bsl], ubeta_v, sem),
    ]
    for c in idx_copies:
        c.wait()
    for j in range(n_chunks):
        sl = pl.ds(j * IDX_CHUNK, IDX_CHUNK)
        uj = uidx_v.at[j]
        ij = iidx_v.at[j]
        copies.append(pltpu.async_copy(ub_hbm.at[uj], ub_v.at[sl], sem))
        copies.append(pltpu.async_copy(vb_hbm.at[ij], vb_v.at[sl], sem))
        copies.append(pltpu.async_copy(ut1_hbm.at[uj], ut1_v.at[sl], sem))
    for c in copies:
        c.wait()

    @plsc.parallel_loop(0, b_per_w // L, unroll=2)
    def group(g):
        off = pl.multiple_of(g * L, L)
        rows = off + lax.iota(jnp.int32, 16)

        acc = jnp.zeros((L,), jnp.float32)
        for d in range(D):
            di = jnp.full((L,), d, jnp.int32)
            a = plsc.load_gather(uf_v, [rows, di])
            b = plsc.load_gather(vf_v, [rows, di])
            acc = acc + a * b

        ub = ub_v[pl.ds(off, L)]
        vb = vb_v[pl.ds(off, L)]
        ut1 = ut1_v[pl.ds(off, L)]

        y = acc + vb + ub

        T = [ut1]
        for k in range(NT - 1):
            bk = plsc.load_gather(ubeta_v, [rows, jnp.full((L,), k, jnp.int32)])
            T.append(T[-1] + jnp.exp(bk))

        one = jnp.ones((L,), jnp.float32)
        sig = [one / (one + jnp.exp(y - t)) for t in T]

        mass = [sig[0]]
        for k in range(1, NT):
            mass.append(sig[k] - sig[k - 1])
        mass.append(one - sig[NT - 1])

        mean = jnp.zeros((L,), jnp.float32)
        best = mass[0]
        bestk = jnp.zeros((L,), jnp.float32)
        for k in range(NBINS):
            mean = mean + mass[k] * (MINR + k * BIN)
            if k > 0:
                gt = mass[k] > best
                best = jnp.where(gt, mass[k], best)
                bestk = jnp.where(gt, jnp.full((L,), float(k), jnp.float32),
                                  bestk)
        mode = MINR + bestk * BIN

        for k in range(NT):
            plsc.store_scatter(edges_v, [rows, jnp.full((L,), k, jnp.int32)],
                               T[k])
        plsc.store_scatter(edges_v, [rows, jnp.full((L,), NT, jnp.int32)],
                           jnp.full((L,), jnp.inf, jnp.float32))
        for k in range(NBINS):
            plsc.store_scatter(mass_v, [rows, jnp.full((L,), k, jnp.int32)],
                               mass[k])
        mean_v[pl.ds(off, L)] = mean
        mode_v[pl.ds(off, L)] = mode

    pltpu.sync_copy(mass_v, mass_hbm.at[bsl])
    pltpu.sync_copy(mean_v, mean_hbm.at[bsl])
    pltpu.sync_copy(mode_v, mode_hbm.at[bsl])
    pltpu.sync_copy(edges_v, edges_hbm.at[bsl])


def kernel(uid_input, iid_input, uid_features, iid_features, uid_bias,
           iid_bias, uid_t1, iid_t1, uid_beta, iid_beta):
    del iid_t1, iid_beta
    B = uid_input.shape[0]
    b_per_w = B // NW
    n_chunks = b_per_w // IDX_CHUNK
    f32 = jnp.float32
    mesh = plsc.VectorSubcoreMesh(core_axis_name="c", subcore_axis_name="s")

    uf_g = uid_features.at[uid_input].get(mode="promise_in_bounds")
    vf_g = iid_features.at[iid_input].get(mode="promise_in_bounds")
    ubeta_g = uid_beta.at[uid_input].get(mode="promise_in_bounds")

    run = pl.kernel(
        functools.partial(_ordrec_body, b_per_w, n_chunks),
        mesh=mesh,
        out_type=[
            jax.ShapeDtypeStruct((B, NBINS), f32),
            jax.ShapeDtypeStruct((B,), f32),
            jax.ShapeDtypeStruct((B,), f32),
            jax.ShapeDtypeStruct((B, NBINS), f32),
        ],
        scratch_types=[
            pltpu.VMEM((n_chunks, IDX_CHUNK), jnp.int32),
            pltpu.VMEM((n_chunks, IDX_CHUNK), jnp.int32),
            pltpu.VMEM((b_per_w, D), f32),
            pltpu.VMEM((b_per_w, D), f32),
            pltpu.VMEM((b_per_w,), f32),
            pltpu.VMEM((b_per_w,), f32),
            pltpu.VMEM((b_per_w,), f32),
            pltpu.VMEM((b_per_w, NT - 1), f32),
            pltpu.VMEM((b_per_w, NBINS), f32),
            pltpu.VMEM((b_per_w,), f32),
            pltpu.VMEM((b_per_w,), f32),
            pltpu.VMEM((b_per_w, NBINS), f32),
            pltpu.SemaphoreType.DMA,
            pltpu.SemaphoreType.DMA,
        ],
        compiler_params=pltpu.CompilerParams(
            use_tc_tiling_on_sc=False, needs_layout_passes=False),
    )
    bins_mass, bins_mean, bins_mode, edges = run(
        uid_input, iid_input, uf_g, vf_g,
        uid_bias.reshape(-1), iid_bias.reshape(-1), uid_t1.reshape(-1),
        ubeta_g)
    return bins_mass, bins_mean, bins_mode, edges
